# baseline (speedup 1.0000x reference)
import jax
import jax.numpy as jnp
from jax import lax
from jax.experimental import pallas as pl
from jax.experimental.pallas import tpu as pltpu
from jax.experimental.pallas import tpu_sc as plsc

NC, NS, L = 2, 16, 16
LAT, LON = 361, 720
NVAR_IN, NVAR_OUT = 73, 10
PER_SC = NVAR_OUT // NC


def _gather_body(x_hbm, idx_hbm, out_hbm, vidx, shared, sem):
    c = lax.axis_index("c")
    s = lax.axis_index("s")
    pltpu.sync_copy(idx_hbm, vidx.at[pl.ds(0, NVAR_OUT)])

    @pl.when(s < PER_SC)
    def _():
        v = s * NC + c
        lane = lax.iota(jnp.int32, L)
        sv = jnp.sum(jnp.where(lane == v, vidx[...], 0))
        pltpu.async_copy(x_hbm.at[0, sv], shared.at[s], sem).wait()
        pltpu.async_copy(shared.at[s], out_hbm.at[0, v], sem).wait()


@jax.jit
def kernel(x, indexes):
    mesh = plsc.VectorSubcoreMesh(
        core_axis_name="c", subcore_axis_name="s", num_cores=NC, num_subcores=NS
    )
    return pl.kernel(
        _gather_body,
        out_type=jax.ShapeDtypeStruct((1, NVAR_OUT, LAT, LON), jnp.float32),
        mesh=mesh,
        scratch_types=[
            pltpu.VMEM((L,), jnp.int32),
            pltpu.VMEM_SHARED((PER_SC, LAT, LON), jnp.float32),
            pltpu.SemaphoreType.DMA,
        ],
        compiler_params=pltpu.CompilerParams(
            needs_layout_passes=False, skip_device_barrier=True
        ),
    )(x, indexes)


# speedup vs baseline: 1.0018x; 1.0018x over previous
import jax
import jax.numpy as jnp
from jax import lax
from jax.experimental import pallas as pl
from jax.experimental.pallas import tpu as pltpu
from jax.experimental.pallas import tpu_sc as plsc

NC, NS, L = 2, 16, 16
LAT, LON = 361, 720
NVAR_IN, NVAR_OUT = 73, 10
PER_SC = NVAR_OUT // NC


def _gather_body(x_hbm, idx_hbm, out_hbm, vidx, shared, sem):
    c = lax.axis_index("c")
    s = lax.axis_index("s")
    pltpu.sync_copy(idx_hbm, vidx.at[pl.ds(0, NVAR_OUT)])

    @pl.when(s < PER_SC)
    def _():
        v = s * NC + c
        lane = lax.iota(jnp.int32, L)
        sv = jnp.sum(jnp.where(lane == v, vidx[...], 0))
        pltpu.async_copy(x_hbm.at[0, sv], shared.at[s], sem).wait()
        pltpu.async_copy(shared.at[s], out_hbm.at[0, v], sem).wait()


@jax.jit
def kernel(x, indexes):
    mesh = plsc.VectorSubcoreMesh(
        core_axis_name="c", subcore_axis_name="s", num_cores=NC, num_subcores=NS
    )
    return pl.kernel(
        _gather_body,
        out_type=jax.ShapeDtypeStruct((1, NVAR_OUT, LAT, LON), jnp.float32),
        mesh=mesh,
        scratch_types=[
            pltpu.VMEM((L,), jnp.int32),
            pltpu.VMEM_SHARED((PER_SC, LAT, LON), jnp.float32),
            pltpu.SemaphoreType.DMA,
        ],
        compiler_params=pltpu.CompilerParams(
            use_tc_tiling_on_sc=True, needs_layout_passes=False
        ),
    )(x, indexes)


# speedup vs baseline: 1.1957x; 1.1936x over previous
import jax
import jax.numpy as jnp
from jax.experimental import pallas as pl
from jax.experimental.pallas import tpu as pltpu

LAT, LON = 361, 720
NVAR_IN, NVAR_OUT = 73, 10


def _copy_body(idx_ref, x_ref, out_ref, *scratch):
    bufs = scratch[:NVAR_OUT]
    insems = scratch[NVAR_OUT:2 * NVAR_OUT]
    outsems = scratch[2 * NVAR_OUT:]
    in_cps = []
    for v in range(NVAR_OUT):
        cp = pltpu.make_async_copy(x_ref.at[0, idx_ref[v]], bufs[v], insems[v])
        cp.start()
        in_cps.append(cp)
    out_cps = []
    for v in range(NVAR_OUT):
        in_cps[v].wait()
        cp = pltpu.make_async_copy(bufs[v], out_ref.at[0, v], outsems[v])
        cp.start()
        out_cps.append(cp)
    for cp in out_cps:
        cp.wait()


@jax.jit
def kernel(x, indexes):
    grid_spec = pltpu.PrefetchScalarGridSpec(
        num_scalar_prefetch=1,
        in_specs=[pl.BlockSpec(memory_space=pltpu.MemorySpace.HBM)],
        out_specs=pl.BlockSpec(memory_space=pltpu.MemorySpace.HBM),
        scratch_shapes=(
            [pltpu.VMEM((LAT, LON), jnp.float32) for _ in range(NVAR_OUT)]
            + [pltpu.SemaphoreType.DMA for _ in range(2 * NVAR_OUT)]
        ),
    )
    return pl.pallas_call(
        _copy_body,
        grid_spec=grid_spec,
        out_shape=jax.ShapeDtypeStruct((1, NVAR_OUT, LAT, LON), jnp.float32),
    )(indexes, x)
